# baseline (speedup 1.0000x reference)
import functools

import jax
import jax.numpy as jnp
import numpy as np
from jax import lax
from jax.experimental import pallas as pl
from jax.experimental.pallas import tpu as pltpu
from jax.experimental.pallas import tpu_sc as plsc

N, E, NREL, D, T = 10000, 320000, 16, 128, 32768
R2 = 2 * NREL
NC, NS = 2, 16
NW = NC * NS
EPW = E // NW
PADW = 20480
NCH = PADW // 128
CH = 64
GW = 40
ACC_ROWS = 10240
TPW = T // NW
TCH = TPW // 128

_mesh = plsc.VectorSubcoreMesh(core_axis_name="c", subcore_axis_name="s")


@functools.partial(
    pl.kernel,
    mesh=_mesh,
    out_type=[
        jax.ShapeDtypeStruct((NW * PADW,), jnp.int32),
        jax.ShapeDtypeStruct((NW * PADW,), jnp.int32),
    ],
    scratch_types=[
        pltpu.VMEM((EPW,), jnp.int32),
        pltpu.VMEM((EPW,), jnp.int32),
        pltpu.VMEM((EPW,), jnp.int32),
        pltpu.VMEM((PADW,), jnp.int32),
        pltpu.VMEM((PADW,), jnp.int32),
    ],
)
def _prep_kernel(s_hbm, r_hbm, o_hbm, gidx_hbm, gdst_hbm, sv, rv, ov, gi, gd):
    wid = lax.axis_index("s") * NC + lax.axis_index("c")
    base = wid * EPW
    pltpu.sync_copy(s_hbm.at[pl.ds(base, EPW)], sv)
    pltpu.sync_copy(r_hbm.at[pl.ds(base, EPW)], rv)
    pltpu.sync_copy(o_hbm.at[pl.ds(base, EPW)], ov)

    def body(i, _):
        sl = pl.ds(i * 16, 16)
        svv = sv[sl]
        rvv = rv[sl]
        ovv = ov[sl]
        gi[sl] = rvv * N + svv
        gd[sl] = ovv
        sl2 = pl.ds(EPW + i * 16, 16)
        gi[sl2] = (rvv + NREL) * N + ovv
        gd[sl2] = svv
        return 0

    lax.fori_loop(0, EPW // 16, body, 0)

    def pad_body(i, _):
        sl = pl.ds(2 * EPW + i * 16, 16)
        gi[sl] = jnp.zeros((16,), jnp.int32)
        gd[sl] = jnp.full((16,), N, jnp.int32)
        return 0

    lax.fori_loop(0, (PADW - 2 * EPW) // 16, pad_body, 0)

    pltpu.sync_copy(gi, gidx_hbm.at[pl.ds(wid * PADW, PADW)])
    pltpu.sync_copy(gd, gdst_hbm.at[pl.ds(wid * PADW, PADW)])


@functools.partial(
    pl.kernel,
    mesh=_mesh,
    out_type=[
        jax.ShapeDtypeStruct((NC, ACC_ROWS, D), jnp.float32),
    ],
    scratch_types=[
        pltpu.VMEM_SHARED((ACC_ROWS, D), jnp.float32),
        pltpu.VMEM((2, GW, CH), jnp.int32),
        pltpu.VMEM((2, GW, CH), jnp.int32),
        pltpu.VMEM((CH, D), jnp.float32),
        pltpu.VMEM((CH, D), jnp.float32),
        pltpu.VMEM((CH, D), jnp.float32),
        pltpu.SemaphoreType.DMA,
        pltpu.SemaphoreType.DMA,
        pltpu.SemaphoreType.DMA,
        pltpu.SemaphoreType.DMA,
        pltpu.SemaphoreType.DMA,
        pltpu.SemaphoreType.DMA,
        pltpu.SemaphoreType.DMA,
    ],
)
def _edge_kernel(y_hbm, gidx_hbm, gdst_hbm, zrows_hbm,
                 acc_hbm,
                 acc_sh, giw, gdw, rb0, rb1, rb2,
                 gs0, gs1, gs2, ss0, ss1, ss2, wsem):
    cid = lax.axis_index("c")
    sid = lax.axis_index("s")
    wid = sid * NC + cid

    rpt = ACC_ROWS // NS
    pltpu.sync_copy(zrows_hbm.at[pl.ds(sid * rpt, rpt)],
                    acc_sh.at[pl.ds(sid * rpt, rpt)])
    plsc.subcore_barrier()

    rbufs = (rb0, rb1, rb2)
    gsems = (gs0, gs1, gs2)
    ssems = (ss0, ss1, ss2)

    def fire_g(wb, c):
        pltpu.async_copy(y_hbm.at[giw.at[wb, c]], rbufs[c % 3], gsems[c % 3])

    def wait_g(c):
        pltpu.make_async_copy(y_hbm.at[giw.at[0, 0]],
                              rbufs[c % 3], gsems[c % 3]).wait()

    def fire_s(wb, c):
        pltpu.async_copy(rbufs[c % 3], acc_sh.at[gdw.at[wb, c]],
                         ssems[c % 3], add=True)

    def wait_s(c):
        pltpu.make_async_copy(rbufs[c % 3], acc_sh.at[gdw.at[0, 0]],
                              ssems[c % 3]).wait()

    NG = (PADW // CH) // GW

    def load_window(gk, wb, sync):
        if sync:
            pltpu.sync_copy(gidx_hbm.at[wid, pl.ds(gk * GW, GW)],
                            giw.at[wb])
            pltpu.sync_copy(gdst_hbm.at[wid, pl.ds(gk * GW, GW)],
                            gdw.at[wb])
        else:
            pltpu.async_copy(gidx_hbm.at[wid, pl.ds(gk * GW, GW)],
                             giw.at[wb], wsem)
            pltpu.async_copy(gdst_hbm.at[wid, pl.ds(gk * GW, GW)],
                             gdw.at[wb], wsem)

    def wait_window():
        pltpu.make_async_copy(gidx_hbm.at[wid, pl.ds(0, GW)],
                              giw.at[0], wsem).wait()
        pltpu.make_async_copy(gdst_hbm.at[wid, pl.ds(0, GW)],
                              gdw.at[0], wsem).wait()

    load_window(0, 0, sync=True)

    def group(gk, _):
        wb = lax.rem(gk, 2)
        fire_g(wb, 0)
        fire_g(wb, 1)

        @pl.when(gk + 1 < NG)
        def _():
            load_window(gk + 1, 1 - wb, sync=False)

        for c in range(GW):
            wait_g(c)
            fire_s(wb, c)
            if c >= 1:
                wait_s(c - 1)
            if c + 2 < GW:
                fire_g(wb, c + 2)
        wait_s(GW - 1)

        @pl.when(gk + 1 < NG)
        def _():
            wait_window()

        return 0

    lax.fori_loop(0, NG, group, 0)

    plsc.subcore_barrier()
    pltpu.sync_copy(acc_sh.at[pl.ds(sid * rpt, rpt)],
                    acc_hbm.at[cid, pl.ds(sid * rpt, rpt)])


@functools.partial(
    pl.kernel,
    mesh=_mesh,
    out_type=[
        jax.ShapeDtypeStruct((NC, 1, ACC_ROWS), jnp.float32),
    ],
    scratch_types=[
        pltpu.VMEM_SHARED((ACC_ROWS,), jnp.float32),
        pltpu.VMEM((NCH, 128), jnp.int32),
        pltpu.VMEM((128,), jnp.float32),
    ],
)
def _deg_kernel(gdst_hbm, zcol_hbm, deg_hbm, deg_sh, gd_v, ones_v):
    cid = lax.axis_index("c")
    sid = lax.axis_index("s")
    wid = sid * NC + cid

    @pl.when(sid == 0)
    def _():
        pltpu.sync_copy(zcol_hbm, deg_sh)

    for k in range(8):
        ones_v[pl.ds(k * 16, 16)] = jnp.ones((16,), jnp.float32)

    pltpu.sync_copy(gdst_hbm.at[wid], gd_v)
    plsc.subcore_barrier()

    def body(g, _):
        pltpu.sync_copy(ones_v, deg_sh.at[gd_v.at[g]], add=True)
        return 0

    lax.fori_loop(0, NCH, body, 0)
    plsc.subcore_barrier()

    @pl.when(sid == 0)
    def _():
        pltpu.sync_copy(deg_sh, deg_hbm.at[cid, 0])


@functools.partial(
    pl.kernel,
    mesh=_mesh,
    out_type=[
        jax.ShapeDtypeStruct((T, D), jnp.float32),
        jax.ShapeDtypeStruct((T, D), jnp.float32),
    ],
    scratch_types=[
        pltpu.VMEM((TCH, 128), jnp.int32),
        pltpu.VMEM((TCH, 128), jnp.int32),
        pltpu.VMEM((128, D), jnp.float32),
        pltpu.VMEM((128, D), jnp.float32),
        pltpu.SemaphoreType.DMA,
        pltpu.SemaphoreType.DMA,
    ],
)
def _tgather_kernel(x_hbm, st_hbm, ot_hbm, a_hbm, b_hbm,
                    st_v, ot_v, rba, rbb, sema, semb):
    wid = lax.axis_index("s") * NC + lax.axis_index("c")
    pltpu.sync_copy(st_hbm.at[wid], st_v)
    pltpu.sync_copy(ot_hbm.at[wid], ot_v)
    base = wid * TPW
    for j in range(TCH):
        pltpu.async_copy(x_hbm.at[st_v.at[j]], rba, sema)
        pltpu.async_copy(x_hbm.at[ot_v.at[j]], rbb, semb)
        pltpu.make_async_copy(x_hbm.at[st_v.at[j]], rba, sema).wait()
        pltpu.sync_copy(rba, a_hbm.at[pl.ds(base + j * 128, 128)])
        pltpu.make_async_copy(x_hbm.at[ot_v.at[j]], rbb, semb).wait()
        pltpu.sync_copy(rbb, b_hbm.at[pl.ds(base + j * 128, 128)])


_BN = 2000
_NB = N // _BN


def _enc_body(emb, w, b, out):
    out[...] = jnp.dot(emb[...], w[...],
                       preferred_element_type=jnp.float32) + b[...]


def _encoder(emb, w, b):
    return pl.pallas_call(
        _enc_body,
        grid=(_NB,),
        in_specs=[
            pl.BlockSpec((_BN, D), lambda i: (i, 0)),
            pl.BlockSpec((D, D), lambda i: (0, 0)),
            pl.BlockSpec((1, D), lambda i: (0, 0)),
        ],
        out_specs=pl.BlockSpec((_BN, D), lambda i: (i, 0)),
        out_shape=jax.ShapeDtypeStruct((N, D), jnp.float32),
    )(emb, w, b)


def _ymm_body(x, w, y):
    y[...] = jnp.dot(x[...].astype(jnp.bfloat16), w[0].astype(jnp.bfloat16),
                     preferred_element_type=jnp.float32)


def _ymm(x, wstack):
    nrel = wstack.shape[0]
    return pl.pallas_call(
        _ymm_body,
        grid=(_NB, nrel),
        in_specs=[
            pl.BlockSpec((_BN, D), lambda i, r: (i, 0)),
            pl.BlockSpec((1, D, D), lambda i, r: (r, 0, 0)),
        ],
        out_specs=pl.BlockSpec((_BN, D), lambda i, r: (r * _NB + i, 0)),
        out_shape=jax.ShapeDtypeStruct((nrel * N, D), jnp.float32),
    )(x, wstack)


def _combine_body(a0, a1, xin, ws, d0, d1, b, out, *, relu):
    invd = 1.0 / (d0[...] + d1[...] + 1.0)
    selfc = jnp.dot(xin[...], ws[...], preferred_element_type=jnp.float32)
    v = (a0[...] + a1[...] + selfc) * invd + b[...]
    if relu:
        v = jnp.maximum(v, 0.0)
    out[...] = v


def _combine(acc0, acc1, xin, wself, d0, d1, b, relu):
    return pl.pallas_call(
        functools.partial(_combine_body, relu=relu),
        grid=(_NB,),
        in_specs=[
            pl.BlockSpec((_BN, D), lambda i: (i, 0)),
            pl.BlockSpec((_BN, D), lambda i: (i, 0)),
            pl.BlockSpec((_BN, D), lambda i: (i, 0)),
            pl.BlockSpec((D, D), lambda i: (0, 0)),
            pl.BlockSpec((_BN, 1), lambda i: (i, 0)),
            pl.BlockSpec((_BN, 1), lambda i: (i, 0)),
            pl.BlockSpec((1, D), lambda i: (0, 0)),
        ],
        out_specs=pl.BlockSpec((_BN, D), lambda i: (i, 0)),
        out_shape=jax.ShapeDtypeStruct((N, D), jnp.float32),
    )(acc0, acc1, xin, wself, d0, d1, b)


def _dec_body(a0, a1, xin, ws, d0, d1, b2, emb, dw, db, out):
    invd = 1.0 / (d0[...] + d1[...] + 1.0)
    selfc = jnp.dot(xin[...], ws[...], preferred_element_type=jnp.float32)
    t = (a0[...] + a1[...] + selfc) * invd + b2[...]
    out[...] = emb[...] + jnp.dot(t, dw[...],
                                  preferred_element_type=jnp.float32) + db[...]


def _decoder(acc0, acc1, xin, wself, d0, d1, b2, emb, dec_W, dec_b):
    return pl.pallas_call(
        _dec_body,
        grid=(_NB,),
        in_specs=[
            pl.BlockSpec((_BN, D), lambda i: (i, 0)),
            pl.BlockSpec((_BN, D), lambda i: (i, 0)),
            pl.BlockSpec((_BN, D), lambda i: (i, 0)),
            pl.BlockSpec((D, D), lambda i: (0, 0)),
            pl.BlockSpec((_BN, 1), lambda i: (i, 0)),
            pl.BlockSpec((_BN, 1), lambda i: (i, 0)),
            pl.BlockSpec((1, D), lambda i: (0, 0)),
            pl.BlockSpec((_BN, D), lambda i: (i, 0)),
            pl.BlockSpec((D, D), lambda i: (0, 0)),
            pl.BlockSpec((1, D), lambda i: (0, 0)),
        ],
        out_specs=pl.BlockSpec((_BN, D), lambda i: (i, 0)),
        out_shape=jax.ShapeDtypeStruct((N, D), jnp.float32),
    )(acc0, acc1, xin, wself, d0, d1, b2, emb, dec_W, dec_b)


_BT = 2048


def _score_body(a, b, p, rel, out):
    onehot = (p[...] == lax.broadcasted_iota(jnp.int32, (1, NREL), 1)
              ).astype(jnp.float32)
    relp = jnp.dot(onehot, rel[...], preferred_element_type=jnp.float32)
    out[...] = jnp.sum(a[...] * relp * b[...], axis=1, keepdims=True)


def _score(a, b, pt, relations):
    return pl.pallas_call(
        _score_body,
        grid=(T // _BT,),
        in_specs=[
            pl.BlockSpec((_BT, D), lambda i: (i, 0)),
            pl.BlockSpec((_BT, D), lambda i: (i, 0)),
            pl.BlockSpec((_BT, 1), lambda i: (i, 0)),
            pl.BlockSpec((NREL, D), lambda i: (0, 0)),
        ],
        out_specs=pl.BlockSpec((_BT, 1), lambda i: (i, 0)),
        out_shape=jax.ShapeDtypeStruct((T, 1), jnp.float32),
    )(a, b, pt, relations)


def kernel(graph, all_triples, node_embeddings, enc_W, enc_b,
           rgc1_W, rgc1_b, rgc2_W, rgc2_b, dec_W, dec_b, relations):
    s = graph[:, 0].astype(jnp.int32)
    r = graph[:, 1].astype(jnp.int32)
    o = graph[:, 2].astype(jnp.int32)

    gidx, gdst = _prep_kernel(s, r, o)
    gidx3 = gidx.reshape(NW, PADW // CH, CH)
    gdst3 = gdst.reshape(NW, PADW // CH, CH)
    gdst3d = gdst.reshape(NW, NCH, 128)

    zrows = jnp.zeros((ACC_ROWS, D), jnp.float32)
    zcol = jnp.zeros((ACC_ROWS,), jnp.float32)

    x0 = _encoder(node_embeddings, enc_W, enc_b.reshape(1, D))

    w1 = rgc1_W[:R2]
    w2 = rgc2_W[:R2]

    y1 = _ymm(x0, w1)
    (deg2,) = _deg_kernel(gdst3d, zcol)
    (acc1,) = _edge_kernel(y1, gidx3, gdst3, zrows)
    d0 = deg2[0, 0, :N].reshape(N, 1)
    d1 = deg2[1, 0, :N].reshape(N, 1)
    x1 = _combine(acc1[0, :N], acc1[1, :N], x0, rgc1_W[R2], d0, d1,
                  rgc1_b.reshape(1, D), relu=True)

    y2 = _ymm(x1, w2)
    (acc2,) = _edge_kernel(y2, gidx3, gdst3, zrows)
    x2 = _decoder(acc2[0, :N], acc2[1, :N], x1, rgc2_W[R2], d0, d1,
                  rgc2_b.reshape(1, D), node_embeddings, dec_W,
                  dec_b.reshape(1, D))

    st = all_triples[:, 0].astype(jnp.int32).reshape(NW, TCH, 128)
    pt = all_triples[:, 1].astype(jnp.int32).reshape(T, 1)
    ot = all_triples[:, 2].astype(jnp.int32).reshape(NW, TCH, 128)
    a_rows, b_rows = _tgather_kernel(x2, st, ot)
    scores = _score(a_rows, b_rows, pt, relations)
    return scores.reshape(-1)


# speedup vs baseline: 1.0073x; 1.0073x over previous
import functools

import jax
import jax.numpy as jnp
import numpy as np
from jax import lax
from jax.experimental import pallas as pl
from jax.experimental.pallas import tpu as pltpu
from jax.experimental.pallas import tpu_sc as plsc

N, E, NREL, D, T = 10000, 320000, 16, 128, 32768
R2 = 2 * NREL
NC, NS = 2, 16
NW = NC * NS
EPW = E // NW
PADW = 20480
NCH = PADW // 128
CH = 64
GW = 40
ACC_ROWS = 10240
TPW = T // NW
TCH = TPW // 128

_mesh = plsc.VectorSubcoreMesh(core_axis_name="c", subcore_axis_name="s")


@functools.partial(
    pl.kernel,
    mesh=_mesh,
    out_type=[
        jax.ShapeDtypeStruct((NW * PADW,), jnp.int32),
        jax.ShapeDtypeStruct((NW * PADW,), jnp.int32),
    ],
    scratch_types=[
        pltpu.VMEM((EPW,), jnp.int32),
        pltpu.VMEM((EPW,), jnp.int32),
        pltpu.VMEM((EPW,), jnp.int32),
        pltpu.VMEM((PADW,), jnp.int32),
        pltpu.VMEM((PADW,), jnp.int32),
    ],
)
def _prep_kernel(s_hbm, r_hbm, o_hbm, gidx_hbm, gdst_hbm, sv, rv, ov, gi, gd):
    wid = lax.axis_index("s") * NC + lax.axis_index("c")
    base = wid * EPW
    pltpu.sync_copy(s_hbm.at[pl.ds(base, EPW)], sv)
    pltpu.sync_copy(r_hbm.at[pl.ds(base, EPW)], rv)
    pltpu.sync_copy(o_hbm.at[pl.ds(base, EPW)], ov)

    def body(i, _):
        sl = pl.ds(i * 16, 16)
        svv = sv[sl]
        rvv = rv[sl]
        ovv = ov[sl]
        gi[sl] = rvv * N + svv
        gd[sl] = ovv
        sl2 = pl.ds(EPW + i * 16, 16)
        gi[sl2] = (rvv + NREL) * N + ovv
        gd[sl2] = svv
        return 0

    lax.fori_loop(0, EPW // 16, body, 0)

    def pad_body(i, _):
        sl = pl.ds(2 * EPW + i * 16, 16)
        gi[sl] = jnp.zeros((16,), jnp.int32)
        gd[sl] = jnp.full((16,), N, jnp.int32)
        return 0

    lax.fori_loop(0, (PADW - 2 * EPW) // 16, pad_body, 0)

    pltpu.sync_copy(gi, gidx_hbm.at[pl.ds(wid * PADW, PADW)])
    pltpu.sync_copy(gd, gdst_hbm.at[pl.ds(wid * PADW, PADW)])


@functools.partial(
    pl.kernel,
    mesh=_mesh,
    out_type=[
        jax.ShapeDtypeStruct((NC, ACC_ROWS, D), jnp.float32),
    ],
    scratch_types=[
        pltpu.VMEM_SHARED((ACC_ROWS, D), jnp.float32),
        pltpu.VMEM((2, GW, CH), jnp.int32),
        pltpu.VMEM((2, GW, CH), jnp.int32),
        pltpu.VMEM((CH, D), jnp.float32),
        pltpu.VMEM((CH, D), jnp.float32),
        pltpu.VMEM((CH, D), jnp.float32),
        pltpu.SemaphoreType.DMA,
        pltpu.SemaphoreType.DMA,
        pltpu.SemaphoreType.DMA,
        pltpu.SemaphoreType.DMA,
        pltpu.SemaphoreType.DMA,
        pltpu.SemaphoreType.DMA,
        pltpu.SemaphoreType.DMA,
    ],
)
def _edge_kernel(y_hbm, gidx_hbm, gdst_hbm, zrows_hbm,
                 acc_hbm,
                 acc_sh, giw, gdw, rb0, rb1, rb2,
                 gs0, gs1, gs2, ss0, ss1, ss2, wsem):
    cid = lax.axis_index("c")
    sid = lax.axis_index("s")
    wid = sid * NC + cid

    rpt = ACC_ROWS // NS
    pltpu.sync_copy(zrows_hbm.at[pl.ds(sid * rpt, rpt)],
                    acc_sh.at[pl.ds(sid * rpt, rpt)])
    plsc.subcore_barrier()

    rbufs = (rb0, rb1, rb2)
    gsems = (gs0, gs1, gs2)
    ssems = (ss0, ss1, ss2)

    def fire_g(wb, c):
        pltpu.async_copy(y_hbm.at[giw.at[wb, c]], rbufs[c % 3], gsems[c % 3])

    def wait_g(c):
        pltpu.make_async_copy(y_hbm.at[giw.at[0, 0]],
                              rbufs[c % 3], gsems[c % 3]).wait()

    def fire_s(wb, c):
        pltpu.async_copy(rbufs[c % 3], acc_sh.at[gdw.at[wb, c]],
                         ssems[c % 3], add=True)

    def wait_s(c):
        pltpu.make_async_copy(rbufs[c % 3], acc_sh.at[gdw.at[0, 0]],
                              ssems[c % 3]).wait()

    NG = (PADW // CH) // GW

    def load_window(gk, wb, sync):
        if sync:
            pltpu.sync_copy(gidx_hbm.at[wid, pl.ds(gk * GW, GW)],
                            giw.at[wb])
            pltpu.sync_copy(gdst_hbm.at[wid, pl.ds(gk * GW, GW)],
                            gdw.at[wb])
        else:
            pltpu.async_copy(gidx_hbm.at[wid, pl.ds(gk * GW, GW)],
                             giw.at[wb], wsem)
            pltpu.async_copy(gdst_hbm.at[wid, pl.ds(gk * GW, GW)],
                             gdw.at[wb], wsem)

    def wait_window():
        pltpu.make_async_copy(gidx_hbm.at[wid, pl.ds(0, GW)],
                              giw.at[0], wsem).wait()
        pltpu.make_async_copy(gdst_hbm.at[wid, pl.ds(0, GW)],
                              gdw.at[0], wsem).wait()

    load_window(0, 0, sync=True)

    def group(gk, _):
        wb = lax.rem(gk, 2)
        fire_g(wb, 0)
        fire_g(wb, 1)

        @pl.when(gk + 1 < NG)
        def _():
            load_window(gk + 1, 1 - wb, sync=False)

        for c in range(GW):
            wait_g(c)
            fire_s(wb, c)
            if c >= 1:
                wait_s(c - 1)
            if c + 2 < GW:
                fire_g(wb, c + 2)
        wait_s(GW - 1)

        @pl.when(gk + 1 < NG)
        def _():
            wait_window()

        return 0

    lax.fori_loop(0, NG, group, 0)

    plsc.subcore_barrier()
    pltpu.sync_copy(acc_sh.at[pl.ds(sid * rpt, rpt)],
                    acc_hbm.at[cid, pl.ds(sid * rpt, rpt)])


@functools.partial(
    pl.kernel,
    mesh=_mesh,
    out_type=[
        jax.ShapeDtypeStruct((NC, 1, ACC_ROWS), jnp.float32),
    ],
    scratch_types=[
        pltpu.VMEM_SHARED((ACC_ROWS,), jnp.float32),
        pltpu.VMEM((NCH, 128), jnp.int32),
        pltpu.VMEM((128,), jnp.float32),
    ],
)
def _deg_kernel(gdst_hbm, zcol_hbm, deg_hbm, deg_sh, gd_v, ones_v):
    cid = lax.axis_index("c")
    sid = lax.axis_index("s")
    wid = sid * NC + cid

    @pl.when(sid == 0)
    def _():
        pltpu.sync_copy(zcol_hbm, deg_sh)

    for k in range(8):
        ones_v[pl.ds(k * 16, 16)] = jnp.ones((16,), jnp.float32)

    pltpu.sync_copy(gdst_hbm.at[wid], gd_v)
    plsc.subcore_barrier()

    def body(g, _):
        pltpu.sync_copy(ones_v, deg_sh.at[gd_v.at[g]], add=True)
        return 0

    lax.fori_loop(0, NCH, body, 0)
    plsc.subcore_barrier()

    @pl.when(sid == 0)
    def _():
        pltpu.sync_copy(deg_sh, deg_hbm.at[cid, 0])


@functools.partial(
    pl.kernel,
    mesh=_mesh,
    out_type=[
        jax.ShapeDtypeStruct((T, D), jnp.float32),
        jax.ShapeDtypeStruct((T, D), jnp.float32),
    ],
    scratch_types=[
        pltpu.VMEM((TCH, 128), jnp.int32),
        pltpu.VMEM((TCH, 128), jnp.int32),
        pltpu.VMEM((128, D), jnp.float32),
        pltpu.VMEM((128, D), jnp.float32),
        pltpu.SemaphoreType.DMA,
        pltpu.SemaphoreType.DMA,
    ],
)
def _tgather_kernel(x_hbm, st_hbm, ot_hbm, a_hbm, b_hbm,
                    st_v, ot_v, rba, rbb, sema, semb):
    wid = lax.axis_index("s") * NC + lax.axis_index("c")
    pltpu.sync_copy(st_hbm.at[wid], st_v)
    pltpu.sync_copy(ot_hbm.at[wid], ot_v)
    base = wid * TPW
    for j in range(TCH):
        pltpu.async_copy(x_hbm.at[st_v.at[j]], rba, sema)
        pltpu.async_copy(x_hbm.at[ot_v.at[j]], rbb, semb)
        pltpu.make_async_copy(x_hbm.at[st_v.at[j]], rba, sema).wait()
        pltpu.sync_copy(rba, a_hbm.at[pl.ds(base + j * 128, 128)])
        pltpu.make_async_copy(x_hbm.at[ot_v.at[j]], rbb, semb).wait()
        pltpu.sync_copy(rbb, b_hbm.at[pl.ds(base + j * 128, 128)])


_BN = 2000
_NB = N // _BN


def _enc_body(emb, w, b, out):
    out[...] = jnp.dot(emb[...], w[...],
                       preferred_element_type=jnp.float32) + b[...]


def _encoder(emb, w, b):
    return pl.pallas_call(
        _enc_body,
        grid=(_NB,),
        in_specs=[
            pl.BlockSpec((_BN, D), lambda i: (i, 0)),
            pl.BlockSpec((D, D), lambda i: (0, 0)),
            pl.BlockSpec((1, D), lambda i: (0, 0)),
        ],
        out_specs=pl.BlockSpec((_BN, D), lambda i: (i, 0)),
        out_shape=jax.ShapeDtypeStruct((N, D), jnp.float32),
    )(emb, w, b)


def _ymm_body(x, w, y):
    y[...] = jnp.dot(x[...], w[0], preferred_element_type=jnp.float32)


def _ymm(x, wstack):
    nrel = wstack.shape[0]
    return pl.pallas_call(
        _ymm_body,
        grid=(_NB, nrel),
        in_specs=[
            pl.BlockSpec((_BN, D), lambda i, r: (i, 0)),
            pl.BlockSpec((1, D, D), lambda i, r: (r, 0, 0)),
        ],
        out_specs=pl.BlockSpec((_BN, D), lambda i, r: (r * _NB + i, 0)),
        out_shape=jax.ShapeDtypeStruct((nrel * N, D), jnp.float32),
    )(x, wstack)


def _combine_body(a0, a1, xin, ws, d0, d1, b, out, *, relu):
    invd = 1.0 / (d0[...] + d1[...] + 1.0)
    selfc = jnp.dot(xin[...], ws[...], preferred_element_type=jnp.float32)
    v = (a0[...] + a1[...] + selfc) * invd + b[...]
    if relu:
        v = jnp.maximum(v, 0.0)
    out[...] = v


def _combine(acc0, acc1, xin, wself, d0, d1, b, relu):
    return pl.pallas_call(
        functools.partial(_combine_body, relu=relu),
        grid=(_NB,),
        in_specs=[
            pl.BlockSpec((_BN, D), lambda i: (i, 0)),
            pl.BlockSpec((_BN, D), lambda i: (i, 0)),
            pl.BlockSpec((_BN, D), lambda i: (i, 0)),
            pl.BlockSpec((D, D), lambda i: (0, 0)),
            pl.BlockSpec((_BN, 1), lambda i: (i, 0)),
            pl.BlockSpec((_BN, 1), lambda i: (i, 0)),
            pl.BlockSpec((1, D), lambda i: (0, 0)),
        ],
        out_specs=pl.BlockSpec((_BN, D), lambda i: (i, 0)),
        out_shape=jax.ShapeDtypeStruct((N, D), jnp.float32),
    )(acc0, acc1, xin, wself, d0, d1, b)


def _dec_body(a0, a1, xin, ws, d0, d1, b2, emb, dw, db, out):
    invd = 1.0 / (d0[...] + d1[...] + 1.0)
    selfc = jnp.dot(xin[...], ws[...], preferred_element_type=jnp.float32)
    t = (a0[...] + a1[...] + selfc) * invd + b2[...]
    out[...] = emb[...] + jnp.dot(t, dw[...],
                                  preferred_element_type=jnp.float32) + db[...]


def _decoder(acc0, acc1, xin, wself, d0, d1, b2, emb, dec_W, dec_b):
    return pl.pallas_call(
        _dec_body,
        grid=(_NB,),
        in_specs=[
            pl.BlockSpec((_BN, D), lambda i: (i, 0)),
            pl.BlockSpec((_BN, D), lambda i: (i, 0)),
            pl.BlockSpec((_BN, D), lambda i: (i, 0)),
            pl.BlockSpec((D, D), lambda i: (0, 0)),
            pl.BlockSpec((_BN, 1), lambda i: (i, 0)),
            pl.BlockSpec((_BN, 1), lambda i: (i, 0)),
            pl.BlockSpec((1, D), lambda i: (0, 0)),
            pl.BlockSpec((_BN, D), lambda i: (i, 0)),
            pl.BlockSpec((D, D), lambda i: (0, 0)),
            pl.BlockSpec((1, D), lambda i: (0, 0)),
        ],
        out_specs=pl.BlockSpec((_BN, D), lambda i: (i, 0)),
        out_shape=jax.ShapeDtypeStruct((N, D), jnp.float32),
    )(acc0, acc1, xin, wself, d0, d1, b2, emb, dec_W, dec_b)


_BT = 2048


def _score_body(a, b, p, rel, out):
    onehot = (p[...] == lax.broadcasted_iota(jnp.int32, (1, NREL), 1)
              ).astype(jnp.float32)
    relp = jnp.dot(onehot, rel[...], preferred_element_type=jnp.float32)
    out[...] = jnp.sum(a[...] * relp * b[...], axis=1, keepdims=True)


def _score(a, b, pt, relations):
    return pl.pallas_call(
        _score_body,
        grid=(T // _BT,),
        in_specs=[
            pl.BlockSpec((_BT, D), lambda i: (i, 0)),
            pl.BlockSpec((_BT, D), lambda i: (i, 0)),
            pl.BlockSpec((_BT, 1), lambda i: (i, 0)),
            pl.BlockSpec((NREL, D), lambda i: (0, 0)),
        ],
        out_specs=pl.BlockSpec((_BT, 1), lambda i: (i, 0)),
        out_shape=jax.ShapeDtypeStruct((T, 1), jnp.float32),
    )(a, b, pt, relations)


def kernel(graph, all_triples, node_embeddings, enc_W, enc_b,
           rgc1_W, rgc1_b, rgc2_W, rgc2_b, dec_W, dec_b, relations):
    s = graph[:, 0].astype(jnp.int32)
    r = graph[:, 1].astype(jnp.int32)
    o = graph[:, 2].astype(jnp.int32)

    gidx, gdst = _prep_kernel(s, r, o)
    gidx3 = gidx.reshape(NW, PADW // CH, CH)
    gdst3 = gdst.reshape(NW, PADW // CH, CH)
    gdst3d = gdst.reshape(NW, NCH, 128)

    zrows = jnp.zeros((ACC_ROWS, D), jnp.float32)
    zcol = jnp.zeros((ACC_ROWS,), jnp.float32)

    x0 = _encoder(node_embeddings, enc_W, enc_b.reshape(1, D))

    w1 = rgc1_W[:R2]
    w2 = rgc2_W[:R2]

    y1 = _ymm(x0, w1)
    (deg2,) = _deg_kernel(gdst3d, zcol)
    (acc1,) = _edge_kernel(y1, gidx3, gdst3, zrows)
    d0 = deg2[0, 0, :N].reshape(N, 1)
    d1 = deg2[1, 0, :N].reshape(N, 1)
    x1 = _combine(acc1[0, :N], acc1[1, :N], x0, rgc1_W[R2], d0, d1,
                  rgc1_b.reshape(1, D), relu=True)

    y2 = _ymm(x1, w2)
    (acc2,) = _edge_kernel(y2, gidx3, gdst3, zrows)
    x2 = _decoder(acc2[0, :N], acc2[1, :N], x1, rgc2_W[R2], d0, d1,
                  rgc2_b.reshape(1, D), node_embeddings, dec_W,
                  dec_b.reshape(1, D))

    st = all_triples[:, 0].astype(jnp.int32).reshape(NW, TCH, 128)
    pt = all_triples[:, 1].astype(jnp.int32).reshape(T, 1)
    ot = all_triples[:, 2].astype(jnp.int32).reshape(NW, TCH, 128)
    a_rows, b_rows = _tgather_kernel(x2, st, ot)
    scores = _score(a_rows, b_rows, pt, relations)
    return scores.reshape(-1)
